# baseline (speedup 1.0000x reference)
import functools

import jax
import jax.numpy as jnp
from jax import lax
from jax.experimental import pallas as pl
from jax.experimental.pallas import tpu as pltpu
from jax.experimental.pallas import tpu_sc as plsc

N_TABLES = 26
TABLE_SIZE = 100000
EMBED_DIM = 32
BATCH = 16384
TOTAL = BATCH * N_TABLES
NUM_WORKERS = 32
PER_WORKER = TOTAL // NUM_WORKERS
CHUNK = 1664
NUM_CHUNKS = PER_WORKER // CHUNK

_mesh = plsc.VectorSubcoreMesh(core_axis_name="c", subcore_axis_name="s")


@functools.partial(
    pl.kernel,
    mesh=_mesh,
    out_type=jax.ShapeDtypeStruct((TOTAL, EMBED_DIM), jnp.float32),
    scratch_types=[
        pltpu.VMEM((CHUNK,), jnp.int32),
        pltpu.VMEM((CHUNK, EMBED_DIM), jnp.float32),
        pltpu.SemaphoreType.DMA,
    ],
    compiler_params=pltpu.CompilerParams(use_tc_tiling_on_sc=False),
)
def _embed(idx_hbm, table_hbm, out_hbm, idx_v, rows_v, sem):
    wid = lax.axis_index("s") * 2 + lax.axis_index("c")
    base = wid * PER_WORKER

    def chunk_body(c, _):
        cbase = base + c * CHUNK
        pltpu.sync_copy(idx_hbm.at[pl.ds(cbase, CHUNK)], idx_v)

        def off_body(i, _):
            pos = lax.broadcasted_iota(jnp.int32, (16,), 0) + i * 16
            col = lax.rem(pos, N_TABLES)
            idx_v[pl.ds(i * 16, 16)] = (
                idx_v[pl.ds(i * 16, 16)] + col * TABLE_SIZE
            )
            return 0

        lax.fori_loop(0, CHUNK // 16, off_body, 0)
        pltpu.async_copy(table_hbm.at[idx_v], rows_v, sem).wait()
        pltpu.sync_copy(rows_v, out_hbm.at[pl.ds(cbase, CHUNK)])
        return 0

    lax.fori_loop(0, NUM_CHUNKS, chunk_body, 0)


def kernel(indices, embedding_table):
    flat_idx = indices.reshape(-1)
    out = _embed(flat_idx, embedding_table)
    return out.reshape(BATCH, N_TABLES, EMBED_DIM)


# speedup vs baseline: 1.0008x; 1.0008x over previous
import functools

import jax
import jax.numpy as jnp
from jax import lax
from jax.experimental import pallas as pl
from jax.experimental.pallas import tpu as pltpu
from jax.experimental.pallas import tpu_sc as plsc

N_TABLES = 26
TABLE_SIZE = 100000
EMBED_DIM = 32
BATCH = 16384
TOTAL = BATCH * N_TABLES
NUM_WORKERS = 32
PER_WORKER = TOTAL // NUM_WORKERS
CHUNK = 1664
NUM_CHUNKS = PER_WORKER // CHUNK

_mesh = plsc.VectorSubcoreMesh(core_axis_name="c", subcore_axis_name="s")


@functools.partial(
    pl.kernel,
    mesh=_mesh,
    out_type=jax.ShapeDtypeStruct((TOTAL, EMBED_DIM), jnp.float32),
    scratch_types=[
        pltpu.VMEM((PER_WORKER,), jnp.int32),
        pltpu.VMEM((2, CHUNK, EMBED_DIM), jnp.float32),
        pltpu.SemaphoreType.DMA((2,)),
        pltpu.SemaphoreType.DMA((2,)),
    ],
    compiler_params=pltpu.CompilerParams(use_tc_tiling_on_sc=False),
)
def _embed(idx_hbm, table_hbm, out_hbm, idx_v, rows_v, gsem, wsem):
    wid = lax.axis_index("s") * 2 + lax.axis_index("c")
    base = wid * PER_WORKER

    pltpu.sync_copy(idx_hbm.at[pl.ds(base, PER_WORKER)], idx_v)

    init_off = (
        lax.rem(lax.broadcasted_iota(jnp.int32, (16,), 0), N_TABLES)
        * TABLE_SIZE
    )

    def off_body(i, off):
        idx_v[pl.ds(i * 16, 16)] = idx_v[pl.ds(i * 16, 16)] + off
        nxt = off + (16 * TABLE_SIZE)
        return nxt - jnp.where(
            nxt >= N_TABLES * TABLE_SIZE, N_TABLES * TABLE_SIZE, 0
        ).astype(jnp.int32)

    lax.fori_loop(0, PER_WORKER // 16, off_body, init_off)

    def gather(c, buf):
        return pltpu.async_copy(
            table_hbm.at[idx_v.at[pl.ds(c * CHUNK, CHUNK)]],
            rows_v.at[buf],
            gsem.at[buf],
        )

    def writeback(c, buf):
        return pltpu.async_copy(
            rows_v.at[buf],
            out_hbm.at[pl.ds(base + c * CHUNK, CHUNK)],
            wsem.at[buf],
        )

    gather(0, 0)
    for c in range(NUM_CHUNKS):
        buf = c % 2
        pltpu.make_async_copy(
            table_hbm.at[idx_v.at[pl.ds(c * CHUNK, CHUNK)]],
            rows_v.at[buf],
            gsem.at[buf],
        ).wait()
        wb = writeback(c, buf)
        if c + 1 < NUM_CHUNKS:
            if c >= 1:
                pltpu.make_async_copy(
                    rows_v.at[(c + 1) % 2],
                    out_hbm.at[pl.ds(base + (c - 1) * CHUNK, CHUNK)],
                    wsem.at[(c + 1) % 2],
                ).wait()
            gather(c + 1, (c + 1) % 2)
    pltpu.make_async_copy(
        rows_v.at[(NUM_CHUNKS - 2) % 2],
        out_hbm.at[pl.ds(base + (NUM_CHUNKS - 2) * CHUNK, CHUNK)],
        wsem.at[(NUM_CHUNKS - 2) % 2],
    ).wait()
    pltpu.make_async_copy(
        rows_v.at[(NUM_CHUNKS - 1) % 2],
        out_hbm.at[pl.ds(base + (NUM_CHUNKS - 1) * CHUNK, CHUNK)],
        wsem.at[(NUM_CHUNKS - 1) % 2],
    ).wait()


def kernel(indices, embedding_table):
    flat_idx = indices.reshape(-1)
    out = _embed(flat_idx, embedding_table)
    return out.reshape(BATCH, N_TABLES, EMBED_DIM)


# speedup vs baseline: 1.0044x; 1.0036x over previous
import functools

import jax
import jax.numpy as jnp
from jax import lax
from jax.experimental import pallas as pl
from jax.experimental.pallas import tpu as pltpu
from jax.experimental.pallas import tpu_sc as plsc

N_TABLES = 26
TABLE_SIZE = 100000
EMBED_DIM = 32
BATCH = 16384
TOTAL = BATCH * N_TABLES
NUM_WORKERS = 32
PER_WORKER = TOTAL // NUM_WORKERS
NBUF = 4
CHUNK = 832
NUM_CHUNKS = PER_WORKER // CHUNK

_mesh = plsc.VectorSubcoreMesh(core_axis_name="c", subcore_axis_name="s")


@functools.partial(
    pl.kernel,
    mesh=_mesh,
    out_type=jax.ShapeDtypeStruct((TOTAL, EMBED_DIM), jnp.float32),
    scratch_types=[
        pltpu.VMEM((PER_WORKER,), jnp.int32),
        pltpu.VMEM((NBUF, CHUNK, EMBED_DIM), jnp.float32),
        pltpu.SemaphoreType.DMA((NBUF,)),
        pltpu.SemaphoreType.DMA((NBUF,)),
    ],
    compiler_params=pltpu.CompilerParams(use_tc_tiling_on_sc=False),
)
def _embed(idx_hbm, table_hbm, out_hbm, idx_v, rows_v, gsem, wsem):
    wid = lax.axis_index("s") * 2 + lax.axis_index("c")
    base = wid * PER_WORKER

    pltpu.sync_copy(idx_hbm.at[pl.ds(base, PER_WORKER)], idx_v)

    init_off = (
        lax.rem(lax.broadcasted_iota(jnp.int32, (16,), 0), N_TABLES)
        * TABLE_SIZE
    )

    def off_body(i, off):
        idx_v[pl.ds(i * 16, 16)] = idx_v[pl.ds(i * 16, 16)] + off
        nxt = off + (16 * TABLE_SIZE)
        return nxt - jnp.where(
            nxt >= N_TABLES * TABLE_SIZE, N_TABLES * TABLE_SIZE, 0
        ).astype(jnp.int32)

    lax.fori_loop(0, PER_WORKER // 16, off_body, init_off)

    def start_gather(c):
        b = c % NBUF
        pltpu.async_copy(
            table_hbm.at[idx_v.at[pl.ds(c * CHUNK, CHUNK)]],
            rows_v.at[b],
            gsem.at[b],
        )

    def wait_gather(c):
        b = c % NBUF
        pltpu.make_async_copy(
            table_hbm.at[idx_v.at[pl.ds(c * CHUNK, CHUNK)]],
            rows_v.at[b],
            gsem.at[b],
        ).wait()

    def start_writeback(c):
        b = c % NBUF
        pltpu.async_copy(
            rows_v.at[b],
            out_hbm.at[pl.ds(base + c * CHUNK, CHUNK)],
            wsem.at[b],
        )

    def wait_writeback(c):
        b = c % NBUF
        pltpu.make_async_copy(
            rows_v.at[b],
            out_hbm.at[pl.ds(base + c * CHUNK, CHUNK)],
            wsem.at[b],
        ).wait()

    for c in range(NBUF - 1):
        start_gather(c)
    for c in range(NUM_CHUNKS):
        wait_gather(c)
        start_writeback(c)
        nxt = c + NBUF - 1
        if nxt < NUM_CHUNKS:
            if c >= 1:
                wait_writeback(nxt - NBUF)
            start_gather(nxt)
    for c in range(NUM_CHUNKS - NBUF, NUM_CHUNKS):
        wait_writeback(c)


def kernel(indices, embedding_table):
    flat_idx = indices.reshape(-1)
    out = _embed(flat_idx, embedding_table)
    return out.reshape(BATCH, N_TABLES, EMBED_DIM)
